# baseline (speedup 1.0000x reference)
import functools

import jax
import jax.numpy as jnp
from jax import lax
from jax.experimental import pallas as pl
from jax.experimental.pallas import tpu as pltpu
from jax.experimental.pallas import tpu_sc as plsc

N = 10000
E = 320000
F1 = 16
F2 = 40
NC = 2
NS = 16
NW = NC * NS
EP = E // NW
C = 125
NCH = EP // C
NP = 10240
RP = NP // NS
DW = 8
NB = 8
LA = 4


def _make_agg(F):
    mesh = plsc.VectorSubcoreMesh(core_axis_name="c", subcore_axis_name="s")

    @functools.partial(
        pl.kernel,
        mesh=mesh,
        compiler_params=pltpu.CompilerParams(use_tc_tiling_on_sc=False),
        out_type=jax.ShapeDtypeStruct((NC * NP, 128), jnp.float32),
        scratch_types=[
            pltpu.VMEM((NCH, C), jnp.int32),
            pltpu.VMEM((NCH, C), jnp.int32),
            pltpu.VMEM((NB, C, F), jnp.float32),
            pltpu.VMEM_SHARED((NP, F), jnp.float32),
            pltpu.SemaphoreType.DMA((NB,)),
            pltpu.SemaphoreType.DMA((NB,)),
        ],
    )
    def agg(ht, srcg, dstg, zrows, out, src_v, dst_v, rows_v, acc, gsem,
            ssem):
        cid = lax.axis_index("c")
        sid = lax.axis_index("s")
        wid = cid * NS + sid
        pltpu.sync_copy(srcg.at[wid], src_v)
        pltpu.sync_copy(dstg.at[wid], dst_v)
        pltpu.sync_copy(zrows, acc.at[pl.ds(sid * RP, RP)])
        plsc.subcore_barrier()

        def gather_wait(jj, b):
            pltpu.make_async_copy(ht.at[src_v.at[jj]], rows_v.at[b],
                                  gsem.at[b]).wait()

        def scatter_wait(jj, b):
            pltpu.make_async_copy(rows_v.at[b], acc.at[dst_v.at[jj]],
                                  ssem.at[b]).wait()

        for k in range(LA):
            pltpu.async_copy(ht.at[src_v.at[k]], rows_v.at[k], gsem.at[k])

        def body(j, carry):
            for b in range(NB):
                jj = j + b
                gather_wait(jj, b)
                pltpu.async_copy(rows_v.at[b], acc.at[dst_v.at[jj]],
                                 ssem.at[b], add=True)
                k = jj + LA
                bk = (b + LA) % NB

                @pl.when(k - NB >= 0)
                def _():
                    scatter_wait(k - NB, bk)

                @pl.when(k < NCH)
                def _():
                    pltpu.async_copy(ht.at[src_v.at[k]], rows_v.at[bk],
                                     gsem.at[bk])
            return carry

        lax.fori_loop(0, NCH // NB, lambda i, c: body(i * NB, c), 0)
        for jj in range(NCH - (NB - LA), NCH):
            scatter_wait(jj, jj % NB)
        plsc.subcore_barrier()
        pltpu.sync_copy(
            acc.at[pl.ds(sid * RP, RP)],
            out.at[pl.ds(cid * NP + sid * RP, RP), pl.ds(0, F)])

    return agg


_deg_mesh = plsc.VectorSubcoreMesh(core_axis_name="c", subcore_axis_name="s")


@functools.partial(
    pl.kernel,
    mesh=_deg_mesh,
    compiler_params=pltpu.CompilerParams(use_tc_tiling_on_sc=False),
    out_type=jax.ShapeDtypeStruct((NC * NP, 128), jnp.float32),
    scratch_types=[
        pltpu.VMEM((NCH, C), jnp.int32),
        pltpu.VMEM((C, DW), jnp.float32),
        pltpu.VMEM_SHARED((NP, DW), jnp.float32),
        pltpu.SemaphoreType.DMA,
    ],
)
def _deg(dstg, ones_rows, zrows, out, dst_v, ones_v, acc, ssem):
    cid = lax.axis_index("c")
    sid = lax.axis_index("s")
    wid = cid * NS + sid
    pltpu.sync_copy(dstg.at[wid], dst_v)
    pltpu.sync_copy(ones_rows, ones_v)
    pltpu.sync_copy(zrows, acc.at[pl.ds(sid * RP, RP)])
    plsc.subcore_barrier()

    def fire(j, carry):
        pltpu.async_copy(ones_v, acc.at[dst_v.at[j]], ssem, add=True)
        return carry

    lax.fori_loop(0, NCH, fire, 0)

    def drain(j, carry):
        pltpu.make_async_copy(ones_v, acc.at[dst_v.at[j]], ssem).wait()
        return carry

    lax.fori_loop(0, NCH, drain, 0)
    plsc.subcore_barrier()
    pltpu.sync_copy(
        acc.at[pl.ds(sid * RP, RP)],
        out.at[pl.ds(cid * NP + sid * RP, RP), pl.ds(0, DW)])


_agg_f1 = _make_agg(F1)
_agg_f2 = _make_agg(F2)


def _tc0_body(x_ref, w1_ref, h_ref):
    h_ref[...] = jnp.dot(x_ref[...], w1_ref[...],
                         preferred_element_type=jnp.float32)


_tc0 = pl.pallas_call(
    _tc0_body,
    out_shape=jax.ShapeDtypeStruct((N, F1), jnp.float32),
)


def _tc1_body(h_ref, degp_ref, ht_ref, dinv_ref):
    deg = (degp_ref[0:N, 0:1] + degp_ref[NP:NP + N, 0:1]
           + 1.0)
    dinv = lax.rsqrt(deg)
    ht_ref[...] = h_ref[...] * dinv
    dinv_ref[...] = dinv


_tc1 = pl.pallas_call(
    _tc1_body,
    out_shape=(jax.ShapeDtypeStruct((N, F1), jnp.float32),
               jax.ShapeDtypeStruct((N, 1), jnp.float32)),
)


def _tc2_body(aggp_ref, ht1_ref, dinv_ref, b1_ref, w2_ref, out_ref):
    agg = (aggp_ref[0:N, 0:F1] + aggp_ref[NP:NP + N, 0:F1] + ht1_ref[...])
    z = dinv_ref[...] * agg + b1_ref[...]
    z = jnp.where(z > 0, z, jnp.exp(z) - 1.0)
    h2 = jnp.dot(z, w2_ref[...], preferred_element_type=jnp.float32)
    out_ref[...] = h2 * dinv_ref[...]


_tc2 = pl.pallas_call(
    _tc2_body,
    out_shape=jax.ShapeDtypeStruct((N, F2), jnp.float32),
)


def _tc3_body(aggp_ref, ht2_ref, dinv_ref, b2_ref, out_ref):
    o = (dinv_ref[...] * (aggp_ref[0:N, 0:F2] + aggp_ref[NP:NP + N, 0:F2]
                          + ht2_ref[...]) + b2_ref[...])
    m = jnp.max(o, axis=1, keepdims=True)
    e = o - m
    lse = jnp.log(jnp.sum(jnp.exp(e), axis=1, keepdims=True))
    out_ref[...] = e - lse


_tc3 = pl.pallas_call(
    _tc3_body,
    out_shape=jax.ShapeDtypeStruct((N, F2), jnp.float32),
)


def kernel(x, edge_index, W1, b1, W2, b2):
    ei = edge_index.astype(jnp.int32)
    srcg = ei[0].reshape(NW, NCH, C)
    dstg = ei[1].reshape(NW, NCH, C)
    ones_rows = jnp.ones((C, DW), jnp.float32)
    zd = jnp.zeros((RP, DW), jnp.float32)
    zf1 = jnp.zeros((RP, F1), jnp.float32)
    zf2 = jnp.zeros((RP, F2), jnp.float32)

    h1 = _tc0(x, W1)
    degp = _deg(dstg, ones_rows, zd)
    ht1, dinv = _tc1(h1, degp)
    aggp1 = _agg_f1(ht1, srcg, dstg, zf1)
    ht2 = _tc2(aggp1, ht1, dinv, b1.reshape(1, F1), W2)
    aggp2 = _agg_f2(ht2, srcg, dstg, zf2)
    return _tc3(aggp2, ht2, dinv, b2.reshape(1, F2))


# speedup vs baseline: 1.0196x; 1.0196x over previous
import functools

import jax
import jax.numpy as jnp
from jax import lax
from jax.experimental import pallas as pl
from jax.experimental.pallas import tpu as pltpu
from jax.experimental.pallas import tpu_sc as plsc

N = 10000
E = 320000
F1 = 16
F2 = 40
NC = 2
NS = 16
NW = NC * NS
EP = E // NW
C = 125
NCH = EP // C
NP = 10240
RP = NP // NS
DW = 8
NB = 8
LA = 4


def _make_agg(F):
    mesh = plsc.VectorSubcoreMesh(core_axis_name="c", subcore_axis_name="s")

    @functools.partial(
        pl.kernel,
        mesh=mesh,
        compiler_params=pltpu.CompilerParams(use_tc_tiling_on_sc=False),
        out_type=jax.ShapeDtypeStruct((NC * NP, 128), jnp.float32),
        scratch_types=[
            pltpu.VMEM((NCH, C), jnp.int32),
            pltpu.VMEM((NCH, C), jnp.int32),
            pltpu.VMEM((NB, C, F), jnp.float32),
            pltpu.VMEM_SHARED((N, F), jnp.float32),
            pltpu.VMEM_SHARED((NP, F), jnp.float32),
            pltpu.SemaphoreType.DMA((NB,)),
            pltpu.SemaphoreType.DMA((NB,)),
        ],
    )
    def agg(ht, srcg, dstg, zrows, out, src_v, dst_v, rows_v, tbl, acc, gsem,
            ssem):
        cid = lax.axis_index("c")
        sid = lax.axis_index("s")
        wid = cid * NS + sid
        pltpu.sync_copy(srcg.at[wid], src_v)
        pltpu.sync_copy(dstg.at[wid], dst_v)
        @pl.when(sid < NS - 1)
        def _():
            pltpu.sync_copy(ht.at[pl.ds(sid * RP, RP)],
                            tbl.at[pl.ds(sid * RP, RP)])

        @pl.when(sid == NS - 1)
        def _():
            pltpu.sync_copy(ht.at[pl.ds((NS - 1) * RP, N - (NS - 1) * RP)],
                            tbl.at[pl.ds((NS - 1) * RP, N - (NS - 1) * RP)])

        pltpu.sync_copy(zrows, acc.at[pl.ds(sid * RP, RP)])
        plsc.subcore_barrier()

        def gather_wait(jj, b):
            pltpu.make_async_copy(tbl.at[src_v.at[jj]], rows_v.at[b],
                                  gsem.at[b]).wait()

        def scatter_wait(jj, b):
            pltpu.make_async_copy(rows_v.at[b], acc.at[dst_v.at[jj]],
                                  ssem.at[b]).wait()

        for k in range(LA):
            pltpu.async_copy(tbl.at[src_v.at[k]], rows_v.at[k], gsem.at[k])

        def body(j, carry):
            for b in range(NB):
                jj = j + b
                gather_wait(jj, b)
                pltpu.async_copy(rows_v.at[b], acc.at[dst_v.at[jj]],
                                 ssem.at[b], add=True)
                k = jj + LA
                bk = (b + LA) % NB

                @pl.when(k - NB >= 0)
                def _():
                    scatter_wait(k - NB, bk)

                @pl.when(k < NCH)
                def _():
                    pltpu.async_copy(tbl.at[src_v.at[k]], rows_v.at[bk],
                                     gsem.at[bk])
            return carry

        lax.fori_loop(0, NCH // NB, lambda i, c: body(i * NB, c), 0)
        for jj in range(NCH - (NB - LA), NCH):
            scatter_wait(jj, jj % NB)
        plsc.subcore_barrier()
        pltpu.sync_copy(
            acc.at[pl.ds(sid * RP, RP)],
            out.at[pl.ds(cid * NP + sid * RP, RP), pl.ds(0, F)])

    return agg


_deg_mesh = plsc.VectorSubcoreMesh(core_axis_name="c", subcore_axis_name="s")


@functools.partial(
    pl.kernel,
    mesh=_deg_mesh,
    compiler_params=pltpu.CompilerParams(use_tc_tiling_on_sc=False),
    out_type=jax.ShapeDtypeStruct((NC * NP, 128), jnp.float32),
    scratch_types=[
        pltpu.VMEM((NCH, C), jnp.int32),
        pltpu.VMEM((C, DW), jnp.float32),
        pltpu.VMEM_SHARED((NP, DW), jnp.float32),
        pltpu.SemaphoreType.DMA,
    ],
)
def _deg(dstg, ones_rows, zrows, out, dst_v, ones_v, acc, ssem):
    cid = lax.axis_index("c")
    sid = lax.axis_index("s")
    wid = cid * NS + sid
    pltpu.sync_copy(dstg.at[wid], dst_v)
    pltpu.sync_copy(ones_rows, ones_v)
    pltpu.sync_copy(zrows, acc.at[pl.ds(sid * RP, RP)])
    plsc.subcore_barrier()

    def fire(j, carry):
        pltpu.async_copy(ones_v, acc.at[dst_v.at[j]], ssem, add=True)
        return carry

    lax.fori_loop(0, NCH, fire, 0)

    def drain(j, carry):
        pltpu.make_async_copy(ones_v, acc.at[dst_v.at[j]], ssem).wait()
        return carry

    lax.fori_loop(0, NCH, drain, 0)
    plsc.subcore_barrier()
    pltpu.sync_copy(
        acc.at[pl.ds(sid * RP, RP)],
        out.at[pl.ds(cid * NP + sid * RP, RP), pl.ds(0, DW)])


_agg_f1 = _make_agg(F1)
_agg_f2 = _make_agg(F2)


def _tc0_body(x_ref, w1_ref, h_ref):
    h_ref[...] = jnp.dot(x_ref[...], w1_ref[...],
                         preferred_element_type=jnp.float32)


_tc0 = pl.pallas_call(
    _tc0_body,
    out_shape=jax.ShapeDtypeStruct((N, F1), jnp.float32),
)


def _tc1_body(h_ref, degp_ref, ht_ref, dinv_ref):
    deg = (degp_ref[0:N, 0:1] + degp_ref[NP:NP + N, 0:1]
           + 1.0)
    dinv = lax.rsqrt(deg)
    ht_ref[...] = h_ref[...] * dinv
    dinv_ref[...] = dinv


_tc1 = pl.pallas_call(
    _tc1_body,
    out_shape=(jax.ShapeDtypeStruct((N, F1), jnp.float32),
               jax.ShapeDtypeStruct((N, 1), jnp.float32)),
)


def _tc2_body(aggp_ref, ht1_ref, dinv_ref, b1_ref, w2_ref, out_ref):
    agg = (aggp_ref[0:N, 0:F1] + aggp_ref[NP:NP + N, 0:F1] + ht1_ref[...])
    z = dinv_ref[...] * agg + b1_ref[...]
    z = jnp.where(z > 0, z, jnp.exp(z) - 1.0)
    h2 = jnp.dot(z, w2_ref[...], preferred_element_type=jnp.float32)
    out_ref[...] = h2 * dinv_ref[...]


_tc2 = pl.pallas_call(
    _tc2_body,
    out_shape=jax.ShapeDtypeStruct((N, F2), jnp.float32),
)


def _tc3_body(aggp_ref, ht2_ref, dinv_ref, b2_ref, out_ref):
    o = (dinv_ref[...] * (aggp_ref[0:N, 0:F2] + aggp_ref[NP:NP + N, 0:F2]
                          + ht2_ref[...]) + b2_ref[...])
    m = jnp.max(o, axis=1, keepdims=True)
    e = o - m
    lse = jnp.log(jnp.sum(jnp.exp(e), axis=1, keepdims=True))
    out_ref[...] = e - lse


_tc3 = pl.pallas_call(
    _tc3_body,
    out_shape=jax.ShapeDtypeStruct((N, F2), jnp.float32),
)


def kernel(x, edge_index, W1, b1, W2, b2):
    ei = edge_index.astype(jnp.int32)
    srcg = ei[0].reshape(NW, NCH, C)
    dstg = ei[1].reshape(NW, NCH, C)
    ones_rows = jnp.ones((C, DW), jnp.float32)
    zd = jnp.zeros((RP, DW), jnp.float32)
    zf1 = jnp.zeros((RP, F1), jnp.float32)
    zf2 = jnp.zeros((RP, F2), jnp.float32)

    h1 = _tc0(x, W1)
    degp = _deg(dstg, ones_rows, zd)
    ht1, dinv = _tc1(h1, degp)
    aggp1 = _agg_f1(ht1, srcg, dstg, zf1)
    ht2 = _tc2(aggp1, ht1, dinv, b1.reshape(1, F1), W2)
    aggp2 = _agg_f2(ht2, srcg, dstg, zf2)
    return _tc3(aggp2, ht2, dinv, b2.reshape(1, F2))


# speedup vs baseline: 1.0458x; 1.0257x over previous
import functools

import jax
import jax.numpy as jnp
from jax import lax
from jax.experimental import pallas as pl
from jax.experimental.pallas import tpu as pltpu
from jax.experimental.pallas import tpu_sc as plsc

N = 10000
E = 320000
F1 = 16
F2 = 40
NC = 2
NS = 16
NW = NC * NS
EP = E // NW
C = 125
NCH = EP // C
NP = 10240
RP = NP // NS
DW = 8
NB = 10
LA = 5


def _make_agg(F, use_tbl):
    mesh = plsc.VectorSubcoreMesh(core_axis_name="c", subcore_axis_name="s")

    @functools.partial(
        pl.kernel,
        mesh=mesh,
        compiler_params=pltpu.CompilerParams(use_tc_tiling_on_sc=False),
        out_type=jax.ShapeDtypeStruct((NC * NP, 128), jnp.float32),
        scratch_types=[
            pltpu.VMEM((NCH, C), jnp.int32),
            pltpu.VMEM((NCH, C), jnp.int32),
            pltpu.VMEM((NB, C, F), jnp.float32),
            pltpu.VMEM_SHARED((N, F) if use_tbl else (8, F), jnp.float32),
            pltpu.VMEM_SHARED((NP, F), jnp.float32),
            pltpu.SemaphoreType.DMA((NB,)),
            pltpu.SemaphoreType.DMA((NB,)),
        ],
    )
    def agg(ht, srcg, dstg, zrows, out, src_v, dst_v, rows_v, tbl_s, acc,
            gsem, ssem):
        cid = lax.axis_index("c")
        sid = lax.axis_index("s")
        wid = cid * NS + sid
        tbl = tbl_s if use_tbl else ht
        pltpu.sync_copy(srcg.at[wid], src_v)
        pltpu.sync_copy(dstg.at[wid], dst_v)
        if use_tbl:
            @pl.when(sid < NS - 1)
            def _():
                pltpu.sync_copy(ht.at[pl.ds(sid * RP, RP)],
                                tbl.at[pl.ds(sid * RP, RP)])

            @pl.when(sid == NS - 1)
            def _():
                pltpu.sync_copy(
                    ht.at[pl.ds((NS - 1) * RP, N - (NS - 1) * RP)],
                    tbl.at[pl.ds((NS - 1) * RP, N - (NS - 1) * RP)])

        pltpu.sync_copy(zrows, acc.at[pl.ds(sid * RP, RP)])
        plsc.subcore_barrier()

        def gather_wait(jj, b):
            pltpu.make_async_copy(tbl.at[src_v.at[jj]], rows_v.at[b],
                                  gsem.at[b]).wait()

        def scatter_wait(jj, b):
            pltpu.make_async_copy(rows_v.at[b], acc.at[dst_v.at[jj]],
                                  ssem.at[b]).wait()

        for k in range(LA):
            pltpu.async_copy(tbl.at[src_v.at[k]], rows_v.at[k], gsem.at[k])

        def body(j, carry):
            for b in range(NB):
                jj = j + b
                gather_wait(jj, b)
                pltpu.async_copy(rows_v.at[b], acc.at[dst_v.at[jj]],
                                 ssem.at[b], add=True)
                k = jj + LA
                bk = (b + LA) % NB

                @pl.when(k - NB >= 0)
                def _():
                    scatter_wait(k - NB, bk)

                @pl.when(k < NCH)
                def _():
                    pltpu.async_copy(tbl.at[src_v.at[k]], rows_v.at[bk],
                                     gsem.at[bk])
            return carry

        lax.fori_loop(0, NCH // NB, lambda i, c: body(i * NB, c), 0)
        for jj in range(NCH - (NB - LA), NCH):
            scatter_wait(jj, jj % NB)
        plsc.subcore_barrier()
        pltpu.sync_copy(
            acc.at[pl.ds(sid * RP, RP)],
            out.at[pl.ds(cid * NP + sid * RP, RP), pl.ds(0, F)])

    return agg


_deg_mesh = plsc.VectorSubcoreMesh(core_axis_name="c", subcore_axis_name="s")


@functools.partial(
    pl.kernel,
    mesh=_deg_mesh,
    compiler_params=pltpu.CompilerParams(use_tc_tiling_on_sc=False),
    out_type=jax.ShapeDtypeStruct((NC * NP, 128), jnp.float32),
    scratch_types=[
        pltpu.VMEM((NCH, C), jnp.int32),
        pltpu.VMEM((C, DW), jnp.float32),
        pltpu.VMEM_SHARED((NP, DW), jnp.float32),
        pltpu.SemaphoreType.DMA,
    ],
)
def _deg(dstg, ones_rows, zrows, out, dst_v, ones_v, acc, ssem):
    cid = lax.axis_index("c")
    sid = lax.axis_index("s")
    wid = cid * NS + sid
    pltpu.sync_copy(dstg.at[wid], dst_v)
    pltpu.sync_copy(ones_rows, ones_v)
    pltpu.sync_copy(zrows, acc.at[pl.ds(sid * RP, RP)])
    plsc.subcore_barrier()

    def fire(j, carry):
        pltpu.async_copy(ones_v, acc.at[dst_v.at[j]], ssem, add=True)
        return carry

    lax.fori_loop(0, NCH, fire, 0)

    def drain(j, carry):
        pltpu.make_async_copy(ones_v, acc.at[dst_v.at[j]], ssem).wait()
        return carry

    lax.fori_loop(0, NCH, drain, 0)
    plsc.subcore_barrier()
    pltpu.sync_copy(
        acc.at[pl.ds(sid * RP, RP)],
        out.at[pl.ds(cid * NP + sid * RP, RP), pl.ds(0, DW)])


_agg_f1 = _make_agg(F1, use_tbl=True)
_agg_f2 = _make_agg(F2, use_tbl=False)


def _tc0_body(x_ref, w1_ref, h_ref):
    h_ref[...] = jnp.dot(x_ref[...], w1_ref[...],
                         preferred_element_type=jnp.float32)


_tc0 = pl.pallas_call(
    _tc0_body,
    out_shape=jax.ShapeDtypeStruct((N, F1), jnp.float32),
)


def _tc1_body(h_ref, degp_ref, ht_ref, dinv_ref):
    deg = (degp_ref[0, 0:N, 0:1] + degp_ref[1, 0:N, 0:1]
           + 1.0)
    dinv = lax.rsqrt(deg)
    ht_ref[...] = h_ref[...] * dinv
    dinv_ref[...] = dinv


_tc1 = pl.pallas_call(
    _tc1_body,
    out_shape=(jax.ShapeDtypeStruct((N, F1), jnp.float32),
               jax.ShapeDtypeStruct((N, 1), jnp.float32)),
)


def _tc2_body(aggp_ref, ht1_ref, dinv_ref, b1_ref, w2_ref, out_ref):
    agg = (aggp_ref[0, 0:N, 0:F1] + aggp_ref[1, 0:N, 0:F1]
           + ht1_ref[...])
    z = dinv_ref[...] * agg + b1_ref[...]
    z = jnp.where(z > 0, z, jnp.exp(z) - 1.0)
    h2 = jnp.dot(z, w2_ref[...], preferred_element_type=jnp.float32)
    out_ref[...] = h2 * dinv_ref[...]


_tc2 = pl.pallas_call(
    _tc2_body,
    out_shape=jax.ShapeDtypeStruct((N, F2), jnp.float32),
)


def _tc3_body(aggp_ref, ht2_ref, dinv_ref, b2_ref, out_ref):
    o = (dinv_ref[...] * (aggp_ref[0, 0:N, 0:F2] + aggp_ref[1, 0:N, 0:F2]
                          + ht2_ref[...]) + b2_ref[...])
    m = jnp.max(o, axis=1, keepdims=True)
    e = o - m
    lse = jnp.log(jnp.sum(jnp.exp(e), axis=1, keepdims=True))
    out_ref[...] = e - lse


_tc3 = pl.pallas_call(
    _tc3_body,
    out_shape=jax.ShapeDtypeStruct((N, F2), jnp.float32),
)


def kernel(x, edge_index, W1, b1, W2, b2):
    ei = edge_index.astype(jnp.int32)
    srcg = ei[0].reshape(NW, NCH, C)
    dstg = ei[1].reshape(NW, NCH, C)
    ones_rows = jnp.ones((C, DW), jnp.float32)
    zd = jnp.zeros((RP, DW), jnp.float32)
    zf1 = jnp.zeros((RP, F1), jnp.float32)
    zf2 = jnp.zeros((RP, F2), jnp.float32)

    h1 = _tc0(x, W1)
    degp = _deg(dstg, ones_rows, zd).reshape(2, NP, 128)
    ht1, dinv = _tc1(h1, degp)
    aggp1 = _agg_f1(ht1, srcg, dstg, zf1).reshape(2, NP, 128)
    ht2 = _tc2(aggp1, ht1, dinv, b1.reshape(1, F1), W2)
    aggp2 = _agg_f2(ht2, srcg, dstg, zf2).reshape(2, NP, 128)
    return _tc3(aggp2, ht2, dinv, b2.reshape(1, F2))


# speedup vs baseline: 1.0493x; 1.0033x over previous
import functools

import jax
import jax.numpy as jnp
from jax import lax
from jax.experimental import pallas as pl
from jax.experimental.pallas import tpu as pltpu
from jax.experimental.pallas import tpu_sc as plsc

N = 10000
E = 320000
F1 = 16
F2 = 40
NC = 2
NS = 16
NW = NC * NS
EP = E // NW
C = 125
NCH = EP // C
NP = 10240
RP = NP // NS
DW = 8
NB = 10
LA = 6


def _make_agg(F, use_tbl):
    mesh = plsc.VectorSubcoreMesh(core_axis_name="c", subcore_axis_name="s")

    @functools.partial(
        pl.kernel,
        mesh=mesh,
        compiler_params=pltpu.CompilerParams(use_tc_tiling_on_sc=False),
        out_type=jax.ShapeDtypeStruct((NC * NP, 128), jnp.float32),
        scratch_types=[
            pltpu.VMEM((NCH, C), jnp.int32),
            pltpu.VMEM((NCH, C), jnp.int32),
            pltpu.VMEM((NB, C, F), jnp.float32),
            pltpu.VMEM_SHARED((N, F) if use_tbl else (8, F), jnp.float32),
            pltpu.VMEM_SHARED((NP, F), jnp.float32),
            pltpu.SemaphoreType.DMA((NB,)),
            pltpu.SemaphoreType.DMA((NB,)),
        ],
    )
    def agg(ht, srcg, dstg, zrows, out, src_v, dst_v, rows_v, tbl_s, acc,
            gsem, ssem):
        cid = lax.axis_index("c")
        sid = lax.axis_index("s")
        wid = cid * NS + sid
        tbl = tbl_s if use_tbl else ht
        pltpu.sync_copy(srcg.at[wid], src_v)
        pltpu.sync_copy(dstg.at[wid], dst_v)
        if use_tbl:
            @pl.when(sid < NS - 1)
            def _():
                pltpu.sync_copy(ht.at[pl.ds(sid * RP, RP)],
                                tbl.at[pl.ds(sid * RP, RP)])

            @pl.when(sid == NS - 1)
            def _():
                pltpu.sync_copy(
                    ht.at[pl.ds((NS - 1) * RP, N - (NS - 1) * RP)],
                    tbl.at[pl.ds((NS - 1) * RP, N - (NS - 1) * RP)])

        pltpu.sync_copy(zrows, acc.at[pl.ds(sid * RP, RP)])
        plsc.subcore_barrier()

        def gather_wait(jj, b):
            pltpu.make_async_copy(tbl.at[src_v.at[jj]], rows_v.at[b],
                                  gsem.at[b]).wait()

        def scatter_wait(jj, b):
            pltpu.make_async_copy(rows_v.at[b], acc.at[dst_v.at[jj]],
                                  ssem.at[b]).wait()

        for k in range(LA):
            pltpu.async_copy(tbl.at[src_v.at[k]], rows_v.at[k], gsem.at[k])

        def body(j, carry):
            for b in range(NB):
                jj = j + b
                gather_wait(jj, b)
                pltpu.async_copy(rows_v.at[b], acc.at[dst_v.at[jj]],
                                 ssem.at[b], add=True)
                k = jj + LA
                bk = (b + LA) % NB

                @pl.when(k - NB >= 0)
                def _():
                    scatter_wait(k - NB, bk)

                @pl.when(k < NCH)
                def _():
                    pltpu.async_copy(tbl.at[src_v.at[k]], rows_v.at[bk],
                                     gsem.at[bk])
            return carry

        lax.fori_loop(0, NCH // NB, lambda i, c: body(i * NB, c), 0)
        for jj in range(NCH - (NB - LA), NCH):
            scatter_wait(jj, jj % NB)
        plsc.subcore_barrier()
        pltpu.sync_copy(
            acc.at[pl.ds(sid * RP, RP)],
            out.at[pl.ds(cid * NP + sid * RP, RP), pl.ds(0, F)])

    return agg


_deg_mesh = plsc.VectorSubcoreMesh(core_axis_name="c", subcore_axis_name="s")


@functools.partial(
    pl.kernel,
    mesh=_deg_mesh,
    compiler_params=pltpu.CompilerParams(use_tc_tiling_on_sc=False),
    out_type=jax.ShapeDtypeStruct((NC * NP, 128), jnp.float32),
    scratch_types=[
        pltpu.VMEM((NCH, C), jnp.int32),
        pltpu.VMEM((C, DW), jnp.float32),
        pltpu.VMEM_SHARED((NP, DW), jnp.float32),
        pltpu.SemaphoreType.DMA,
    ],
)
def _deg(dstg, ones_rows, zrows, out, dst_v, ones_v, acc, ssem):
    cid = lax.axis_index("c")
    sid = lax.axis_index("s")
    wid = cid * NS + sid
    pltpu.sync_copy(dstg.at[wid], dst_v)
    pltpu.sync_copy(ones_rows, ones_v)
    pltpu.sync_copy(zrows, acc.at[pl.ds(sid * RP, RP)])
    plsc.subcore_barrier()

    def fire(j, carry):
        pltpu.async_copy(ones_v, acc.at[dst_v.at[j]], ssem, add=True)
        return carry

    lax.fori_loop(0, NCH, fire, 0)

    def drain(j, carry):
        pltpu.make_async_copy(ones_v, acc.at[dst_v.at[j]], ssem).wait()
        return carry

    lax.fori_loop(0, NCH, drain, 0)
    plsc.subcore_barrier()
    pltpu.sync_copy(
        acc.at[pl.ds(sid * RP, RP)],
        out.at[pl.ds(cid * NP + sid * RP, RP), pl.ds(0, DW)])


_agg_f1 = _make_agg(F1, use_tbl=True)
_agg_f2 = _make_agg(F2, use_tbl=False)


def _tc0_body(x_ref, w1_ref, h_ref):
    h_ref[...] = jnp.dot(x_ref[...], w1_ref[...],
                         preferred_element_type=jnp.float32)


_tc0 = pl.pallas_call(
    _tc0_body,
    out_shape=jax.ShapeDtypeStruct((N, F1), jnp.float32),
)


def _tc1_body(h_ref, degp_ref, ht_ref, dinv_ref):
    deg = (degp_ref[0, 0:N, 0:1] + degp_ref[1, 0:N, 0:1]
           + 1.0)
    dinv = lax.rsqrt(deg)
    ht_ref[...] = h_ref[...] * dinv
    dinv_ref[...] = dinv


_tc1 = pl.pallas_call(
    _tc1_body,
    out_shape=(jax.ShapeDtypeStruct((N, F1), jnp.float32),
               jax.ShapeDtypeStruct((N, 1), jnp.float32)),
)


def _tc2_body(aggp_ref, ht1_ref, dinv_ref, b1_ref, w2_ref, out_ref):
    agg = (aggp_ref[0, 0:N, 0:F1] + aggp_ref[1, 0:N, 0:F1]
           + ht1_ref[...])
    z = dinv_ref[...] * agg + b1_ref[...]
    z = jnp.where(z > 0, z, jnp.exp(z) - 1.0)
    h2 = jnp.dot(z, w2_ref[...], preferred_element_type=jnp.float32)
    out_ref[...] = h2 * dinv_ref[...]


_tc2 = pl.pallas_call(
    _tc2_body,
    out_shape=jax.ShapeDtypeStruct((N, F2), jnp.float32),
)


def _tc3_body(aggp_ref, ht2_ref, dinv_ref, b2_ref, out_ref):
    o = (dinv_ref[...] * (aggp_ref[0, 0:N, 0:F2] + aggp_ref[1, 0:N, 0:F2]
                          + ht2_ref[...]) + b2_ref[...])
    m = jnp.max(o, axis=1, keepdims=True)
    e = o - m
    lse = jnp.log(jnp.sum(jnp.exp(e), axis=1, keepdims=True))
    out_ref[...] = e - lse


_tc3 = pl.pallas_call(
    _tc3_body,
    out_shape=jax.ShapeDtypeStruct((N, F2), jnp.float32),
)


def kernel(x, edge_index, W1, b1, W2, b2):
    ei = edge_index.astype(jnp.int32)
    srcg = ei[0].reshape(NW, NCH, C)
    dstg = ei[1].reshape(NW, NCH, C)
    ones_rows = jnp.ones((C, DW), jnp.float32)
    zd = jnp.zeros((RP, DW), jnp.float32)
    zf1 = jnp.zeros((RP, F1), jnp.float32)
    zf2 = jnp.zeros((RP, F2), jnp.float32)

    h1 = _tc0(x, W1)
    degp = _deg(dstg, ones_rows, zd).reshape(2, NP, 128)
    ht1, dinv = _tc1(h1, degp)
    aggp1 = _agg_f1(ht1, srcg, dstg, zf1).reshape(2, NP, 128)
    ht2 = _tc2(aggp1, ht1, dinv, b1.reshape(1, F1), W2)
    aggp2 = _agg_f2(ht2, srcg, dstg, zf2).reshape(2, NP, 128)
    return _tc3(aggp2, ht2, dinv, b2.reshape(1, F2))
